# baseline (speedup 1.0000x reference)
import functools

import jax
import jax.numpy as jnp
from jax import lax
from jax.experimental import pallas as pl
from jax.experimental.pallas import tpu as pltpu
from jax.experimental.pallas import tpu_sc as plsc

_B, _N, _M, _NB = 64, 128, 256, 10
_AF, _BF, _H = 128, 16, 256
_DEPTH = 3

_NW = 32
_MPW = _B // _NW
_AC = 8
_EC = _AC * _NB
_NCHUNK = _N // _AC
_PAD = 8

_f32 = jnp.float32


def _dot(a, b):
    return jax.lax.dot_general(a, b, (((1,), (0,)), ((), ())),
                               preferred_element_type=_f32)


def _tc_pre_body(af_ref, bf_ref, w1_ref, wnh_ref, wnb_ref, bn_ref, w2b_ref,
                 h_ref, hn_ref, gbt_ref, tb2_ref):
    h = jnp.maximum(_dot(af_ref[...], w1_ref[...]), 0.0)
    h_ref[...] = h
    hn_ref[...] = _dot(h, wnh_ref[...])
    gbt_ref[...] = _dot(bf_ref[...], wnb_ref[...]) + bn_ref[...]
    tb2_ref[...] = _dot(bf_ref[...], w2b_ref[...])


def _tc_update_body(h_ref, nei_ref, wah_ref, wan_ref, ba_ref, wnh_ref,
                    h2_ref, hn2_ref, last):
    h2 = jnp.maximum(
        _dot(h_ref[...], wah_ref[...]) + _dot(nei_ref[...], wan_ref[...])
        + ba_ref[...], 0.0)
    h2_ref[...] = h2
    hn2_ref[...] = _dot(h2, wnh_ref[...])


def _tc_final_body(h_ref, nei_ref, matom_ref, w2_ref, out_ref):
    out_ref[...] = _dot(h_ref[...], w2_ref[...]) * nei_ref[...] * matom_ref[...]


_ROWS_PER_STEP = 1024


def _rows_spec(cols):
    return pl.BlockSpec((_ROWS_PER_STEP, cols), lambda i: (i, 0))


def _rep_spec(*blk):
    return pl.BlockSpec(blk, lambda i: (0,) * len(blk))


def _tc_pre(af, bf, W1, wnh, wnb, bn2, W2b):
    grid = (_B * _N // _ROWS_PER_STEP,)
    bond_rows = _B * _M // (_B * _N // _ROWS_PER_STEP)
    return pl.pallas_call(
        _tc_pre_body,
        grid=grid,
        in_specs=[
            _rows_spec(_AF),
            pl.BlockSpec((bond_rows, _BF), lambda i: (i, 0)),
            _rep_spec(_AF, _H), _rep_spec(_H, _H), _rep_spec(_BF, _H),
            _rep_spec(1, _H), _rep_spec(_BF, _H),
        ],
        out_specs=[
            _rows_spec(_H), _rows_spec(_H),
            pl.BlockSpec((bond_rows, _H), lambda i: (i, 0)),
            pl.BlockSpec((bond_rows, _H), lambda i: (i, 0)),
        ],
        out_shape=[
            jax.ShapeDtypeStruct((_B * _N, _H), _f32),
            jax.ShapeDtypeStruct((_B * _N, _H), _f32),
            jax.ShapeDtypeStruct((_B * _M, _H), _f32),
            jax.ShapeDtypeStruct((_B * _M, _H), _f32),
        ],
    )(af, bf, W1, wnh, wnb, bn2, W2b)


def _tc_update(h, nei, wah, wan, ba2, wnext):
    grid = (_B * _N // _ROWS_PER_STEP,)
    return pl.pallas_call(
        functools.partial(_tc_update_body, last=False),
        grid=grid,
        in_specs=[
            _rows_spec(_H), _rows_spec(_H),
            _rep_spec(_H, _H), _rep_spec(_H, _H), _rep_spec(1, _H),
            _rep_spec(_H, _H),
        ],
        out_specs=[_rows_spec(_H), _rows_spec(_H)],
        out_shape=[
            jax.ShapeDtypeStruct((_B * _N, _H), _f32),
            jax.ShapeDtypeStruct((_B * _N, _H), _f32),
        ],
    )(h, nei, wah, wan, ba2, wnext)


def _tc_final(h, nei, matom, W2):
    grid = (_B * _N // _ROWS_PER_STEP,)
    return pl.pallas_call(
        _tc_final_body,
        grid=grid,
        in_specs=[
            _rows_spec(_H), _rows_spec(_H),
            pl.BlockSpec((_ROWS_PER_STEP, 1), lambda i: (i, 0)),
            _rep_spec(_H, _H),
        ],
        out_specs=_rows_spec(_H),
        out_shape=jax.ShapeDtypeStruct((_B * _N, _H), _f32),
    )(h, nei, matom, W2)


def _sc_agg_body(product, atab_hbm, btab_hbm, ia_hbm, ib_hbm, out_hbm,
                 ia_v, ib_v, ga_v, gb_v, nei_v, sema, semb):
    wid = lax.axis_index("s") * 2 + lax.axis_index("c")
    ebase = wid * (_MPW * _N * _NB)
    abase = wid * (_MPW * _N)
    pltpu.sync_copy(ia_hbm.at[pl.ds(ebase, _MPW * _N * _NB)], ia_v)
    pltpu.sync_copy(ib_hbm.at[pl.ds(ebase, _MPW * _N * _NB)], ib_v)

    def chunk_body(c, _):
        e0 = c * _EC
        cpa = pltpu.async_copy(
            atab_hbm.at[ia_v.at[pl.ds(e0, _EC)]], ga_v, sema)
        cpb = pltpu.async_copy(
            btab_hbm.at[ib_v.at[pl.ds(e0, _EC)]], gb_v, semb)
        cpa.wait()
        cpb.wait()

        def atom_body(a, _):
            def vreg_body(k, _):
                def nb_body(nb, acc):
                    e = a * _NB + nb
                    va = ga_v[e, pl.ds(k * 16, 16)]
                    vb = gb_v[e, pl.ds(k * 16, 16)]
                    if product:
                        return acc + va * vb
                    return acc + jnp.maximum(va + vb, 0.0)
                acc = lax.fori_loop(0, _NB, nb_body, jnp.zeros((16,), _f32))
                nei_v[a, pl.ds(k * 16, 16)] = acc
                return 0
            lax.fori_loop(0, _H // 16, vreg_body, 0)
            return 0
        lax.fori_loop(0, _AC, atom_body, 0)
        pltpu.sync_copy(nei_v, out_hbm.at[pl.ds(abase + c * _AC, _AC)])
        return 0

    lax.fori_loop(0, _MPW * _NCHUNK, chunk_body, 0)


def _sc_agg(atab, btab, ia, ib, product):
    mesh = plsc.VectorSubcoreMesh(core_axis_name="c", subcore_axis_name="s")
    kfn = functools.partial(
        pl.kernel,
        mesh=mesh,
        out_type=jax.ShapeDtypeStruct((_B * _N, _H), _f32),
        scratch_types=[
            pltpu.VMEM((_MPW * _N * _NB,), jnp.int32),
            pltpu.VMEM((_MPW * _N * _NB,), jnp.int32),
            pltpu.VMEM((_EC, _H), _f32),
            pltpu.VMEM((_EC, _H), _f32),
            pltpu.VMEM((_AC, _H), _f32),
            pltpu.SemaphoreType.DMA,
            pltpu.SemaphoreType.DMA,
        ],
    )(functools.partial(_sc_agg_body, product))
    return kfn(atab, btab, ia, ib)


@jax.jit
def kernel(atom_feats, bond_feats, atom_graph, bond_graph, num_nbs, n_atoms,
           mask_neis, mask_atoms, W1, Wn, bn, Wa, ba, W2a, W2b, W2):
    del num_nbs, n_atoms
    boff = (jnp.arange(_B, dtype=jnp.int32) * _N)[:, None, None]
    ia = (atom_graph.astype(jnp.int32) + boff).reshape(-1)
    mflat = mask_neis.reshape(_B, _N, _NB)
    boffm = (jnp.arange(_B, dtype=jnp.int32) * _M)[:, None, None]
    ib = jnp.where(mflat, bond_graph.astype(jnp.int32) + boffm,
                   jnp.int32(_B * _M)).reshape(-1)
    matom = mask_atoms.astype(_f32).reshape(_B * _N, 1)

    wnh, wnb = Wn[:_H], Wn[_H:]
    wah, wan = Wa[:_H], Wa[_H:]
    bn2 = bn.reshape(1, _H)
    ba2 = ba.reshape(1, _H)

    af = atom_feats.reshape(_B * _N, _AF)
    bf = bond_feats.reshape(_B * _M, _BF)

    h, hn, gbt, tb2 = _tc_pre(af, bf, W1, wnh, wnb, bn2, W2b)
    gbt = jnp.concatenate(
        [gbt, jnp.full((_PAD, _H), -1e30, _f32)], axis=0)
    tb2 = jnp.concatenate(
        [tb2, jnp.zeros((_PAD, _H), _f32)], axis=0)

    for _ in range(_DEPTH - 2):
        nei = _sc_agg(hn, gbt, ia, ib, product=False)
        h, hn = _tc_update(h, nei, wah, wan, ba2, wnh)
    nei = _sc_agg(hn, gbt, ia, ib, product=False)
    h, ha = _tc_update(h, nei, wah, wan, ba2, W2a)
    nei = _sc_agg(ha, tb2, ia, ib, product=True)
    local = _tc_final(h, nei, matom, W2)
    return local.reshape(_B, _N, _H)
